# baseline (device time: 43809 ns/iter reference)
import os

import jax
import jax.numpy as jnp
from jax import lax
from jax.experimental import pallas as pl
from jax.experimental.pallas import tpu as pltpu

N_DEV = 32
GA = 8
GB = 4
F8 = jnp.float8_e4m3fn

VARIANT = os.environ.get("SCBAND_KVARIANT", "full")


def kernel(x, w_mat, scale_x, scale_w):
    m_tot, k_loc = x.shape
    k_tot, n = w_mat.shape
    m_per = m_tot // N_DEV
    k_blk = k_tot // GA

    def body(x_ref, w_ref, sx_ref, sw_ref, out_ref,
             w_stage, xs_ref, comm_ref,
             w_sems, send_sems, recv_sems, ready_sems):
        my = lax.axis_index("i")
        my_a = my // GB

        def issue_w_dma(j):
            bb = lax.rem(my_a + j, GA)
            dma = pltpu.make_async_copy(
                w_ref.at[pl.ds(bb * k_blk, k_blk), :],
                w_stage.at[pl.ds(bb * k_blk, k_blk), :],
                w_sems.at[j],
            )
            dma.start()
            return dma

        w_dmas = [issue_w_dma(j) for j in range(GA)]

        xs_ref[...] = x_ref[...].astype(F8)

        local = pltpu.make_async_copy(
            xs_ref.at[pl.ds(my * m_per, m_per), :],
            comm_ref.at[my],
            recv_sems.at[my],
        )
        local.start()

        sends = []

        def send_to(dst):
            d = lax.rem(dst - my + N_DEV, N_DEV)
            pl.semaphore_wait(ready_sems.at[dst], 1)
            rdma = pltpu.make_async_remote_copy(
                src_ref=xs_ref.at[pl.ds(dst * m_per, m_per), :],
                dst_ref=comm_ref.at[my],
                send_sem=send_sems.at[d],
                recv_sem=recv_sems.at[my],
                device_id=(dst,),
                device_id_type=pl.DeviceIdType.MESH,
            )
            rdma.start()
            sends.append(rdma)

        if VARIANT == "full":
            for d in range(1, N_DEV):
                pl.semaphore_signal(
                    ready_sems.at[my], inc=1,
                    device_id=(lax.rem(my + d, N_DEV),),
                    device_id_type=pl.DeviceIdType.MESH,
                )
            for db in range(1, GB):
                send_to(my_a * GB + lax.rem(my + db, GB))

        acc = None
        for j in range(GA):
            bb = lax.rem(my_a + j, GA)
            if VARIANT == "full" and j >= 1:
                ga = lax.rem(my_a - j + GA, GA)
                for b2 in range(GB):
                    send_to(ga * GB + b2)

            w_dmas[j].wait()
            if VARIANT == "streamonly":
                continue

            for i in range(GB):
                s = bb * GB + i
                recv = pltpu.make_async_remote_copy(
                    src_ref=xs_ref.at[pl.ds(0, m_per), :],
                    dst_ref=comm_ref.at[s],
                    send_sem=send_sems.at[0],
                    recv_sem=recv_sems.at[s],
                    device_id=(s,),
                    device_id_type=pl.DeviceIdType.MESH,
                )
                recv.wait_recv()

            xg_blk = jnp.concatenate(
                [comm_ref[bb * GB + i] for i in range(GB)],
                axis=1).astype(jnp.float32)
            term = jnp.dot(
                xg_blk,
                w_stage[pl.ds(bb * k_blk, k_blk), :],
                preferred_element_type=jnp.float32,
            )
            acc = term if acc is None else acc + term

        scale = sx_ref[0] * sw_ref[0]
        if acc is None:
            local.wait()
            out_ref[...] = jnp.zeros((m_per, n), jnp.float32) + scale
        else:
            out_ref[...] = jnp.maximum(acc * scale, 0.0)

        for rdma in sends:
            rdma.wait_send()

    return pl.pallas_call(
        body,
        out_shape=jax.ShapeDtypeStruct((m_per, n), jnp.float32),
        in_specs=[
            pl.BlockSpec(memory_space=pltpu.VMEM),
            pl.BlockSpec(memory_space=pl.ANY),
            pl.BlockSpec(memory_space=pltpu.SMEM),
            pl.BlockSpec(memory_space=pltpu.SMEM),
        ],
        out_specs=pl.BlockSpec(memory_space=pltpu.VMEM),
        scratch_shapes=[
            pltpu.VMEM((k_tot, n), jnp.float32),
            pltpu.VMEM((m_tot, k_loc), F8),
            pltpu.VMEM((N_DEV, m_per, k_loc), F8),
            pltpu.SemaphoreType.DMA((GA,)),
            pltpu.SemaphoreType.DMA((N_DEV,)),
            pltpu.SemaphoreType.DMA((N_DEV,)),
            pltpu.SemaphoreType.REGULAR((N_DEV,)),
        ],
        compiler_params=pltpu.CompilerParams(
            vmem_limit_bytes=64 * 1024 * 1024,
            skip_device_barrier=True,
        ),
    )(x, w_mat, scale_x, scale_w)


# device time: 22622 ns/iter; 1.9366x vs baseline; 1.9366x over previous
import os

import jax
import jax.numpy as jnp
from jax import lax
from jax.experimental import pallas as pl
from jax.experimental.pallas import tpu as pltpu

N_DEV = 32
F8 = jnp.float8_e4m3fn

VARIANT = os.environ.get("SCBAND_KVARIANT", "full")
N_WBLK = int(os.environ.get("SCBAND_NWBLK", "8"))
PRE_CHUNKS = int(os.environ.get("SCBAND_PRECHUNK", "3"))


def kernel(x, w_mat, scale_x, scale_w):
    m_tot, k_loc = x.shape
    k_tot, n = w_mat.shape
    m_per = m_tot // N_DEV
    k_blk = k_tot // N_WBLK
    s_per_blk = k_blk // m_per

    def body(x_ref, w_ref, sx_ref, sw_ref, out_ref,
             w_stage, xs_ref, comm_ref,
             w_sems, send_sems, recv_sems, ready_sems):
        my = lax.axis_index("i")
        my_blk = my // s_per_blk

        def issue_w_dma(j):
            bb = lax.rem(my_blk + j, N_WBLK)
            dma = pltpu.make_async_copy(
                w_ref.at[pl.ds(bb * k_blk, k_blk), :],
                w_stage.at[pl.ds(bb * k_blk, k_blk), :],
                w_sems.at[j],
            )
            dma.start()
            return dma

        w_dmas = [issue_w_dma(j) for j in range(PRE_CHUNKS)]

        xs_ref[...] = x_ref[...].astype(F8)
        local = pltpu.make_async_copy(
            xs_ref.at[pl.ds(my * m_per, m_per), :],
            comm_ref.at[my],
            recv_sems.at[my],
        )
        local.start()

        sends = []
        if VARIANT == "full":
            for d in range(1, N_DEV):
                pl.semaphore_signal(
                    ready_sems.at[my], inc=1,
                    device_id=(lax.rem(my + d, N_DEV),),
                    device_id_type=pl.DeviceIdType.MESH,
                )
            for d in range(1, N_DEV):
                dst = lax.rem(my + d, N_DEV)
                pl.semaphore_wait(ready_sems.at[dst], 1)
                rdma = pltpu.make_async_remote_copy(
                    src_ref=xs_ref.at[pl.ds(dst * m_per, m_per), :],
                    dst_ref=comm_ref.at[my],
                    send_sem=send_sems.at[d],
                    recv_sem=recv_sems.at[my],
                    device_id=(dst,),
                    device_id_type=pl.DeviceIdType.MESH,
                )
                rdma.start()
                sends.append(rdma)

        w_dmas += [issue_w_dma(j) for j in range(PRE_CHUNKS, N_WBLK)]

        acc = None
        for j in range(N_WBLK):
            bb = lax.rem(my_blk + j, N_WBLK)
            w_dmas[j].wait()
            if VARIANT == "streamonly":
                continue

            for i in range(s_per_blk):
                s = bb * s_per_blk + i
                recv = pltpu.make_async_remote_copy(
                    src_ref=xs_ref.at[pl.ds(0, m_per), :],
                    dst_ref=comm_ref.at[s],
                    send_sem=send_sems.at[0],
                    recv_sem=recv_sems.at[s],
                    device_id=(s,),
                    device_id_type=pl.DeviceIdType.MESH,
                )
                recv.wait_recv()

            xg_blk = jnp.concatenate(
                [comm_ref[bb * s_per_blk + i] for i in range(s_per_blk)],
                axis=1).astype(jnp.float32)
            term = jnp.dot(
                xg_blk,
                w_stage[pl.ds(bb * k_blk, k_blk), :],
                preferred_element_type=jnp.float32,
            )
            acc = term if acc is None else acc + term

        scale = sx_ref[0] * sw_ref[0]
        if acc is None:
            local.wait()
            out_ref[...] = jnp.zeros((m_per, n), jnp.float32) + scale
        else:
            out_ref[...] = jnp.maximum(acc * scale, 0.0)

        for rdma in sends:
            rdma.wait_send()

    return pl.pallas_call(
        body,
        out_shape=jax.ShapeDtypeStruct((m_per, n), jnp.float32),
        in_specs=[
            pl.BlockSpec(memory_space=pltpu.VMEM),
            pl.BlockSpec(memory_space=pl.ANY),
            pl.BlockSpec(memory_space=pltpu.SMEM),
            pl.BlockSpec(memory_space=pltpu.SMEM),
        ],
        out_specs=pl.BlockSpec(memory_space=pltpu.VMEM),
        scratch_shapes=[
            pltpu.VMEM((k_tot, n), jnp.float32),
            pltpu.VMEM((m_tot, k_loc), F8),
            pltpu.VMEM((N_DEV, m_per, k_loc), F8),
            pltpu.SemaphoreType.DMA((N_WBLK,)),
            pltpu.SemaphoreType.DMA((N_DEV,)),
            pltpu.SemaphoreType.DMA((N_DEV,)),
            pltpu.SemaphoreType.REGULAR((N_DEV,)),
        ],
        compiler_params=pltpu.CompilerParams(
            vmem_limit_bytes=64 * 1024 * 1024,
            skip_device_barrier=True,
        ),
    )(x, w_mat, scale_x, scale_w)
